# baseline (speedup 1.0000x reference)
import functools
import math

import jax
import jax.numpy as jnp
from jax import lax
from jax.experimental import pallas as pl
from jax.experimental.pallas import tpu as pltpu
from jax.experimental.pallas import tpu_sc as plsc

N = 10000
E = 320000
H = 128
HEADS = 4
HD = H // HEADS
FF = 4 * H

BN = 1000
BE = 1280
GN = N // BN
GE = E // BE


def _ln(x, g, b):
    mu = jnp.mean(x, axis=-1, keepdims=True)
    var = jnp.mean((x - mu) ** 2, axis=-1, keepdims=True)
    return (x - mu) * lax.rsqrt(var + 1e-5) * g + b


def _erf(x):
    s = jnp.sign(x)
    a = jnp.abs(x)
    t = 1.0 / (1.0 + 0.3275911 * a)
    poly = t * (0.254829592 + t * (-0.284496736 + t * (1.421413741
               + t * (-1.453152027 + t * 1.061405429))))
    return s * (1.0 - poly * jnp.exp(-a * a))


def _gelu(x):
    return x * 0.5 * (1.0 + _erf(x * (1.0 / math.sqrt(2.0))))


def _k1_body(x_ref, wq_ref, wk_ref, wv_ref, bq_ref, bk_ref, bv_ref,
             xq_out, xk_out, xv_out):
    xb = x_ref[...]
    f32 = jnp.float32
    xq_out[...] = jnp.dot(xb, wq_ref[...], preferred_element_type=f32) + bq_ref[...]
    xk_out[...] = jnp.dot(xb, wk_ref[...], preferred_element_type=f32) + bk_ref[...]
    xv_out[...] = jnp.dot(xb, wv_ref[...], preferred_element_type=f32) + bv_ref[...]


def _k1(x, p):
    spec_n = pl.BlockSpec((BN, H), lambda i: (i, 0))
    spec_w = pl.BlockSpec((H, H), lambda i: (0, 0))
    spec_b = pl.BlockSpec((1, H), lambda i: (0, 0))
    out = pl.pallas_call(
        _k1_body,
        grid=(GN,),
        in_specs=[spec_n, spec_w, spec_w, spec_w, spec_b, spec_b, spec_b],
        out_specs=[spec_n, spec_n, spec_n],
        out_shape=[jax.ShapeDtypeStruct((N, H), jnp.float32)] * 3,
    )(x, p['Wq'], p['Wk'], p['Wv'],
      p['bq'].reshape(1, H), p['bk'].reshape(1, H), p['bv'].reshape(1, H))
    return out


def _k2_body(qd_ref, ks_ref, vs_ref, g_ref, tp_ref, wk_ref, wv_ref,
             wtb_ref, btb_ref, s_out, v_out):
    f32 = jnp.float32
    g = g_ref[...]
    kfull = ks_ref[...] + jnp.dot(g, wk_ref[...], preferred_element_type=f32)
    v_out[...] = vs_ref[...] + jnp.dot(g, wv_ref[...], preferred_element_type=f32)
    prod = qd_ref[...] * kfull
    r = lax.broadcasted_iota(jnp.int32, (H, HEADS), 0) // HD
    c = lax.broadcasted_iota(jnp.int32, (H, HEADS), 1)
    hm = jnp.where(r == c, 1.0, 0.0).astype(f32)
    s = jnp.dot(prod, hm, preferred_element_type=f32) * (1.0 / math.sqrt(HD))
    tb = jnp.tanh(tp_ref[...] * wtb_ref[...] + btb_ref[...])
    s_out[...] = s + tb


def _k2(qd, ks, vs, gate_embed, tpos, p):
    spec_e = pl.BlockSpec((BE, H), lambda i: (i, 0))
    spec_t = pl.BlockSpec((BE, 1), lambda i: (i, 0))
    spec_w = pl.BlockSpec((H, H), lambda i: (0, 0))
    spec_b4 = pl.BlockSpec((1, HEADS), lambda i: (0, 0))
    spec_s = pl.BlockSpec((BE, HEADS), lambda i: (i, 0))
    return pl.pallas_call(
        _k2_body,
        grid=(GE,),
        in_specs=[spec_e, spec_e, spec_e, spec_e, spec_t, spec_w, spec_w,
                  spec_b4, spec_b4],
        out_specs=[spec_s, spec_e],
        out_shape=[jax.ShapeDtypeStruct((E, HEADS), jnp.float32),
                   jax.ShapeDtypeStruct((E, H), jnp.float32)],
    )(qd, ks, vs, gate_embed, tpos.reshape(E, 1), p['Wk'], p['Wv'],
      p['Wtb'], p['btb'].reshape(1, HEADS))


def _k3_body(part_ref, out_ref, *, op):
    x = part_ref[...]
    if op == 'max':
        out_ref[...] = jnp.max(x, axis=0)
    else:
        out_ref[...] = jnp.sum(x, axis=0)


def _k3(part, op):
    bn = 2000
    return pl.pallas_call(
        functools.partial(_k3_body, op=op),
        grid=(N // bn,),
        in_specs=[pl.BlockSpec((32, bn, HEADS), lambda i: (0, i, 0))],
        out_specs=pl.BlockSpec((bn, HEADS), lambda i: (i, 0)),
        out_shape=jax.ShapeDtypeStruct((N, HEADS), jnp.float32),
    )(part)


def _k4_body(aggp_ref, x_ref, wo_ref, bo_ref, lag_ref, lab_ref,
             w1a_ref, w1b_ref, bm1_ref, h_out, a_out, b_out):
    f32 = jnp.float32
    agg = aggp_ref[0] + aggp_ref[1]
    out = jnp.dot(agg, wo_ref[...], preferred_element_type=f32) + bo_ref[...]
    h = _ln(x_ref[...] + out, lag_ref[...], lab_ref[...])
    h_out[...] = h
    a_out[...] = jnp.dot(h, w1a_ref[...], preferred_element_type=f32) + bm1_ref[...]
    b_out[...] = jnp.dot(h, w1b_ref[...], preferred_element_type=f32)


def _k4(agg_part, x, p):
    spec_p = pl.BlockSpec((2, BN, H), lambda i: (0, i, 0))
    spec_n = pl.BlockSpec((BN, H), lambda i: (i, 0))
    spec_w = pl.BlockSpec((H, H), lambda i: (0, 0))
    spec_w2 = pl.BlockSpec((H, 2 * H), lambda i: (0, 0))
    spec_b = pl.BlockSpec((1, H), lambda i: (0, 0))
    spec_b2 = pl.BlockSpec((1, 2 * H), lambda i: (0, 0))
    spec_n2 = pl.BlockSpec((BN, 2 * H), lambda i: (i, 0))
    return pl.pallas_call(
        _k4_body,
        grid=(GN,),
        in_specs=[spec_p, spec_n, spec_w, spec_b, spec_b, spec_b,
                  spec_w2, spec_w2, spec_b2],
        out_specs=[spec_n, spec_n2, spec_n2],
        out_shape=[jax.ShapeDtypeStruct((N, H), jnp.float32),
                   jax.ShapeDtypeStruct((N, 2 * H), jnp.float32),
                   jax.ShapeDtypeStruct((N, 2 * H), jnp.float32)],
    )(agg_part, x, p['Wo'], p['bo'].reshape(1, H),
      p['ln_a_g'].reshape(1, H), p['ln_a_b'].reshape(1, H),
      p['Wm1'][:H], p['Wm1'][H:2 * H], p['bm1'].reshape(1, 2 * H))


def _k5_body(pre_ref, g_ref, qd_ref, w1c_ref, wm2_ref, bm2_ref,
             wd_ref, bd_ref, m_out):
    f32 = jnp.float32
    t = pre_ref[...] + jnp.dot(g_ref[...], w1c_ref[...], preferred_element_type=f32)
    z = jnp.dot(_gelu(t), wm2_ref[...], preferred_element_type=f32) + bm2_ref[...]
    dw = jnp.mean(jax.nn.sigmoid(qd_ref[...] * wd_ref[...] + bd_ref[...]),
                  axis=-1, keepdims=True)
    m_out[...] = z * (1.0 + dw)


def _k5(pre, gate_embed, qdist, p):
    spec_p = pl.BlockSpec((BE, 2 * H), lambda i: (i, 0))
    spec_e = pl.BlockSpec((BE, H), lambda i: (i, 0))
    spec_t = pl.BlockSpec((BE, 1), lambda i: (i, 0))
    spec_w1 = pl.BlockSpec((H, 2 * H), lambda i: (0, 0))
    spec_w2 = pl.BlockSpec((2 * H, H), lambda i: (0, 0))
    spec_bh = pl.BlockSpec((1, H), lambda i: (0, 0))
    spec_b4 = pl.BlockSpec((1, HEADS), lambda i: (0, 0))
    return pl.pallas_call(
        _k5_body,
        grid=(GE,),
        in_specs=[spec_p, spec_e, spec_t, spec_w1, spec_w2, spec_bh,
                  spec_b4, spec_b4],
        out_specs=spec_e,
        out_shape=jax.ShapeDtypeStruct((E, H), jnp.float32),
    )(pre, gate_embed, qdist.reshape(E, 1), p['Wm1'][2 * H:], p['Wm2'],
      p['bm2'].reshape(1, H), p['Wd'], p['bd'].reshape(1, HEADS))


def _k6_body(h_ref, a2p_ref, wg1_ref, wg2_ref, bg_ref, wu1_ref, wu2_ref,
             bu_ref, lcg_ref, lcb_ref, n1g_ref, n1b_ref, wf1_ref, bf1_ref,
             wf2_ref, bf2_ref, wga1_ref, wga2_ref, bga_ref, n2g_ref,
             n2b_ref, out_ref):
    f32 = jnp.float32
    h = h_ref[...]
    agg2 = a2p_ref[0] + a2p_ref[1]
    gate = jax.nn.sigmoid(
        jnp.dot(h, wg1_ref[...], preferred_element_type=f32)
        + jnp.dot(agg2, wg2_ref[...], preferred_element_type=f32) + bg_ref[...])
    upd = _gelu(
        jnp.dot(h, wu1_ref[...], preferred_element_type=f32)
        + jnp.dot(agg2, wu2_ref[...], preferred_element_type=f32) + bu_ref[...])
    hc = _ln(h * (1.0 - gate) + upd * gate, lcg_ref[...], lcb_ref[...])
    hn = _ln(hc, n1g_ref[...], n1b_ref[...])
    hff = jnp.dot(_gelu(jnp.dot(hn, wf1_ref[...], preferred_element_type=f32)
                        + bf1_ref[...]), wf2_ref[...],
                  preferred_element_type=f32) + bf2_ref[...]
    g2 = jax.nn.sigmoid(
        jnp.dot(hc, wga1_ref[...], preferred_element_type=f32)
        + jnp.dot(hff, wga2_ref[...], preferred_element_type=f32) + bga_ref[...])
    out_ref[...] = _ln(hc + g2 * hff, n2g_ref[...], n2b_ref[...])


def _k6(h, agg2_part, p):
    spec_n = pl.BlockSpec((BN, H), lambda i: (i, 0))
    spec_p = pl.BlockSpec((2, BN, H), lambda i: (0, i, 0))
    spec_w = pl.BlockSpec((H, H), lambda i: (0, 0))
    spec_wf1 = pl.BlockSpec((H, FF), lambda i: (0, 0))
    spec_wf2 = pl.BlockSpec((FF, H), lambda i: (0, 0))
    spec_b = pl.BlockSpec((1, H), lambda i: (0, 0))
    spec_bf = pl.BlockSpec((1, FF), lambda i: (0, 0))
    return pl.pallas_call(
        _k6_body,
        grid=(GN,),
        in_specs=[spec_n, spec_p, spec_w, spec_w, spec_b, spec_w, spec_w,
                  spec_b, spec_b, spec_b, spec_b, spec_b, spec_wf1, spec_bf,
                  spec_wf2, spec_b, spec_w, spec_w, spec_b, spec_b, spec_b],
        out_specs=spec_n,
        out_shape=jax.ShapeDtypeStruct((N, H), jnp.float32),
    )(h, agg2_part,
      p['Wg'][:H], p['Wg'][H:], p['bg'].reshape(1, H),
      p['Wu'][:H], p['Wu'][H:], p['bu'].reshape(1, H),
      p['ln_c_g'].reshape(1, H), p['ln_c_b'].reshape(1, H),
      p['n1_g'].reshape(1, H), p['n1_b'].reshape(1, H),
      p['Wf1'], p['bf1'].reshape(1, FF), p['Wf2'], p['bf2'].reshape(1, H),
      p['Wgate'][:H], p['Wgate'][H:], p['bgate'].reshape(1, H),
      p['n2_g'].reshape(1, H), p['n2_b'].reshape(1, H))


def kernel(x, edge_index, gate_embed, temporal_pos, qubit_distance, params):
    p = params
    src = edge_index[0]
    dst = edge_index[1]

    xq, xk, xv = _k1(x, p)

    qd = xq[dst]
    ks = xk[src]
    vs = xv[src]

    scores, vfull = _k2(qd, ks, vs, gate_embed, temporal_pos, p)

    smax = jax.ops.segment_max(scores, dst, num_segments=N)
    smax = jnp.where(jnp.isfinite(smax), smax, 0.0)
    ex = jnp.exp(scores - smax[dst])
    den = jax.ops.segment_sum(ex, dst, num_segments=N)

    attn = ex / (den[dst] + 1e-16)
    msg = jnp.repeat(attn, HD, axis=1) * vfull
    agg = jax.ops.segment_sum(msg, dst, num_segments=N)
    agg_part = jnp.stack([agg, jnp.zeros_like(agg)])

    h, a, b = _k4(agg_part, x, p)

    pre = a[dst] + b[src]

    msg2 = _k5(pre, gate_embed, qubit_distance, p)

    agg2 = jax.ops.segment_sum(msg2, dst, num_segments=N)
    agg2_part = jnp.stack([agg2, jnp.zeros_like(agg2)])

    return _k6(h, agg2_part, p)


# speedup vs baseline: 3.2315x; 3.2315x over previous
import functools
import math

import jax
import jax.numpy as jnp
from jax import lax
from jax.experimental import pallas as pl
from jax.experimental.pallas import tpu as pltpu
from jax.experimental.pallas import tpu_sc as plsc

N = 10000
E = 320000
H = 128
HEADS = 4
HD = H // HEADS
FF = 4 * H

BN = 1000
BE = 1280
GN = N // BN
GE = E // BE

NC = 2
NS = 16
NW = NC * NS
EPW = E // NW
CH = 80
NCH = EPW // CH
RPT = 624
RREM = N - NS * RPT

_MESH = plsc.VectorSubcoreMesh(core_axis_name="c", subcore_axis_name="s")


def _ln(x, g, b):
    mu = jnp.mean(x, axis=-1, keepdims=True)
    var = jnp.mean((x - mu) ** 2, axis=-1, keepdims=True)
    return (x - mu) * lax.rsqrt(var + 1e-5) * g + b


def _erf(x):
    s = jnp.sign(x)
    a = jnp.abs(x)
    t = 1.0 / (1.0 + 0.3275911 * a)
    poly = t * (0.254829592 + t * (-0.284496736 + t * (1.421413741
               + t * (-1.453152027 + t * 1.061405429))))
    return s * (1.0 - poly * jnp.exp(-a * a))


def _gelu(x):
    return x * 0.5 * (1.0 + _erf(x * (1.0 / math.sqrt(2.0))))


def _g1(xq, xk, xv, dst, src):
    @functools.partial(
        pl.kernel,
        mesh=_MESH,
        out_type=[jax.ShapeDtypeStruct((E, H), jnp.float32)] * 3,
        scratch_types=[
            pltpu.VMEM((CH,), jnp.int32),
            pltpu.VMEM((CH,), jnp.int32),
            pltpu.VMEM((CH, H), jnp.float32),
            pltpu.VMEM((CH, H), jnp.float32),
            pltpu.VMEM((CH, H), jnp.float32),
            pltpu.SemaphoreType.DMA,
            pltpu.SemaphoreType.DMA,
            pltpu.SemaphoreType.DMA,
        ],
    )
    def body(xq_h, xk_h, xv_h, dst_h, src_h, qd_o, ks_o, vs_o,
             idxd, idxs, rq, rk, rv, s1, s2, s3):
        wid = lax.axis_index("s") * NC + lax.axis_index("c")
        base = wid * EPW

        def chunk(i, c):
            off = base + i * CH
            pltpu.sync_copy(dst_h.at[pl.ds(off, CH)], idxd)
            pltpu.sync_copy(src_h.at[pl.ds(off, CH)], idxs)
            ca = pltpu.async_copy(xq_h.at[idxd], rq, s1)
            cb = pltpu.async_copy(xk_h.at[idxs], rk, s2)
            cc = pltpu.async_copy(xv_h.at[idxs], rv, s3)
            ca.wait()
            pltpu.sync_copy(rq, qd_o.at[pl.ds(off, CH)])
            cb.wait()
            pltpu.sync_copy(rk, ks_o.at[pl.ds(off, CH)])
            cc.wait()
            pltpu.sync_copy(rv, vs_o.at[pl.ds(off, CH)])
            return c

        lax.fori_loop(0, NCH, chunk, 0)

    return body(xq, xk, xv, dst, src)


def _g2(atab, btab, dst, src):
    @functools.partial(
        pl.kernel,
        mesh=_MESH,
        out_type=[jax.ShapeDtypeStruct((E, 2 * H), jnp.float32)] * 2,
        scratch_types=[
            pltpu.VMEM((CH,), jnp.int32),
            pltpu.VMEM((CH,), jnp.int32),
            pltpu.VMEM((CH, 2 * H), jnp.float32),
            pltpu.VMEM((CH, 2 * H), jnp.float32),
            pltpu.SemaphoreType.DMA,
            pltpu.SemaphoreType.DMA,
        ],
    )
    def body(a_h, b_h, dst_h, src_h, ad_o, bs_o, idxd, idxs, ra, rb, s1, s2):
        wid = lax.axis_index("s") * NC + lax.axis_index("c")
        base = wid * EPW

        def chunk(i, c):
            off = base + i * CH
            pltpu.sync_copy(dst_h.at[pl.ds(off, CH)], idxd)
            pltpu.sync_copy(src_h.at[pl.ds(off, CH)], idxs)
            ca = pltpu.async_copy(a_h.at[idxd], ra, s1)
            cb = pltpu.async_copy(b_h.at[idxs], rb, s2)
            ca.wait()
            pltpu.sync_copy(ra, ad_o.at[pl.ds(off, CH)])
            cb.wait()
            pltpu.sync_copy(rb, bs_o.at[pl.ds(off, CH)])
            return c

        lax.fori_loop(0, NCH, chunk, 0)

    return body(atab, btab, dst, src)


def _sseg(rows_arr, dst, zrows, w):
    @functools.partial(
        pl.kernel,
        mesh=_MESH,
        out_type=jax.ShapeDtypeStruct((2, N, w), jnp.float32),
        scratch_types=[
            pltpu.VMEM((CH,), jnp.int32),
            pltpu.VMEM((CH, w), jnp.float32),
            pltpu.VMEM_SHARED((N, w), jnp.float32),
        ],
    )
    def body(msg_h, dst_h, zm_h, aggp_o, idx, rows, sh_m):
        cid = lax.axis_index("c")
        sid = lax.axis_index("s")
        wid = sid * NC + cid

        @pl.when(sid == 0)
        def _():
            pltpu.sync_copy(zm_h, sh_m)

        plsc.subcore_barrier()
        base = wid * EPW

        def chunk(i, c):
            off = base + i * CH
            pltpu.sync_copy(dst_h.at[pl.ds(off, CH)], idx)
            pltpu.sync_copy(msg_h.at[pl.ds(off, CH)], rows)
            pltpu.sync_copy(rows, sh_m.at[idx], add=True)
            return c

        lax.fori_loop(0, NCH, chunk, 0)
        plsc.subcore_barrier()
        offr = pl.multiple_of(sid * RPT, 8)
        pltpu.sync_copy(sh_m.at[pl.ds(offr, RPT)],
                        aggp_o.at[cid, pl.ds(offr, RPT)])

        @pl.when(sid == 0)
        def _():
            pltpu.sync_copy(sh_m.at[pl.ds(NS * RPT, RREM)],
                            aggp_o.at[cid, pl.ds(NS * RPT, RREM)])

    return body(rows_arr, dst, zrows)


def _k1_body(x_ref, wq_ref, wk_ref, wv_ref, bq_ref, bk_ref, bv_ref,
             xq_out, xk_out, xv_out):
    xb = x_ref[...]
    f32 = jnp.float32
    xq_out[...] = jnp.dot(xb, wq_ref[...], preferred_element_type=f32) + bq_ref[...]
    xk_out[...] = jnp.dot(xb, wk_ref[...], preferred_element_type=f32) + bk_ref[...]
    xv_out[...] = jnp.dot(xb, wv_ref[...], preferred_element_type=f32) + bv_ref[...]


def _k1(x, p):
    spec_n = pl.BlockSpec((BN, H), lambda i: (i, 0))
    spec_w = pl.BlockSpec((H, H), lambda i: (0, 0))
    spec_b = pl.BlockSpec((1, H), lambda i: (0, 0))
    out = pl.pallas_call(
        _k1_body,
        grid=(GN,),
        in_specs=[spec_n, spec_w, spec_w, spec_w, spec_b, spec_b, spec_b],
        out_specs=[spec_n, spec_n, spec_n],
        out_shape=[jax.ShapeDtypeStruct((N, H), jnp.float32)] * 3,
    )(x, p['Wq'], p['Wk'], p['Wv'],
      p['bq'].reshape(1, H), p['bk'].reshape(1, H), p['bv'].reshape(1, H))
    return out


def _k2_body(qd_ref, ks_ref, vs_ref, g_ref, tp_ref, wk_ref, wv_ref,
             wtb_ref, btb_ref, msg_out, exw_out):
    f32 = jnp.float32
    g = g_ref[...]
    kfull = ks_ref[...] + jnp.dot(g, wk_ref[...], preferred_element_type=f32)
    vfull = vs_ref[...] + jnp.dot(g, wv_ref[...], preferred_element_type=f32)
    prod = qd_ref[...] * kfull
    r = lax.broadcasted_iota(jnp.int32, (H, HEADS), 0) // HD
    c = lax.broadcasted_iota(jnp.int32, (H, HEADS), 1)
    hm = jnp.where(r == c, 1.0, 0.0).astype(f32)
    s = jnp.dot(prod, hm, preferred_element_type=f32) * (1.0 / math.sqrt(HD))
    tb = jnp.tanh(tp_ref[...] * wtb_ref[...] + btb_ref[...])
    ex = jnp.exp(s + tb)
    rr = lax.broadcasted_iota(jnp.int32, (HEADS, H), 0)
    cc = lax.broadcasted_iota(jnp.int32, (HEADS, H), 1) // HD
    expand = jnp.where(rr == cc, 1.0, 0.0).astype(f32)
    exw = jnp.dot(ex, expand, preferred_element_type=f32)
    msg_out[...] = exw * vfull
    exw_out[...] = exw


def _k2(qd, ks, vs, gate_embed, tpos, p):
    spec_e = pl.BlockSpec((BE, H), lambda i: (i, 0))
    spec_t = pl.BlockSpec((BE, 1), lambda i: (i, 0))
    spec_w = pl.BlockSpec((H, H), lambda i: (0, 0))
    spec_b4 = pl.BlockSpec((1, HEADS), lambda i: (0, 0))
    return pl.pallas_call(
        _k2_body,
        grid=(GE,),
        in_specs=[spec_e, spec_e, spec_e, spec_e, spec_t, spec_w, spec_w,
                  spec_b4, spec_b4],
        out_specs=[spec_e, spec_e],
        out_shape=[jax.ShapeDtypeStruct((E, H), jnp.float32),
                   jax.ShapeDtypeStruct((E, H), jnp.float32)],
    )(qd, ks, vs, gate_embed, tpos.reshape(E, 1), p['Wk'], p['Wv'],
      p['Wtb'], p['btb'].reshape(1, HEADS))


def _k4_body(aggp_ref, denp_ref, x_ref, wo_ref, bo_ref, lag_ref, lab_ref,
             w1a_ref, w1b_ref, bm1_ref, h_out, a_out, b_out):
    f32 = jnp.float32
    den = denp_ref[0] + denp_ref[1]
    agg = (aggp_ref[0] + aggp_ref[1]) / (den + 1e-16)
    out = jnp.dot(agg, wo_ref[...], preferred_element_type=f32) + bo_ref[...]
    h = _ln(x_ref[...] + out, lag_ref[...], lab_ref[...])
    h_out[...] = h
    a_out[...] = jnp.dot(h, w1a_ref[...], preferred_element_type=f32) + bm1_ref[...]
    b_out[...] = jnp.dot(h, w1b_ref[...], preferred_element_type=f32)


def _k4(agg_part, den_part, x, p):
    spec_p = pl.BlockSpec((2, BN, H), lambda i: (0, i, 0))
    spec_n = pl.BlockSpec((BN, H), lambda i: (i, 0))
    spec_w = pl.BlockSpec((H, H), lambda i: (0, 0))
    spec_w2 = pl.BlockSpec((H, 2 * H), lambda i: (0, 0))
    spec_b = pl.BlockSpec((1, H), lambda i: (0, 0))
    spec_b2 = pl.BlockSpec((1, 2 * H), lambda i: (0, 0))
    spec_n2 = pl.BlockSpec((BN, 2 * H), lambda i: (i, 0))
    return pl.pallas_call(
        _k4_body,
        grid=(GN,),
        in_specs=[spec_p, spec_p, spec_n, spec_w, spec_b, spec_b, spec_b,
                  spec_w2, spec_w2, spec_b2],
        out_specs=[spec_n, spec_n2, spec_n2],
        out_shape=[jax.ShapeDtypeStruct((N, H), jnp.float32),
                   jax.ShapeDtypeStruct((N, 2 * H), jnp.float32),
                   jax.ShapeDtypeStruct((N, 2 * H), jnp.float32)],
    )(agg_part, den_part, x, p['Wo'], p['bo'].reshape(1, H),
      p['ln_a_g'].reshape(1, H), p['ln_a_b'].reshape(1, H),
      p['Wm1'][:H], p['Wm1'][H:2 * H], p['bm1'].reshape(1, 2 * H))


def _k5_body(ad_ref, bs_ref, g_ref, qd_ref, w1c_ref, wm2_ref, bm2_ref,
             wd_ref, bd_ref, m_out):
    f32 = jnp.float32
    t = (ad_ref[...] + bs_ref[...]
         + jnp.dot(g_ref[...], w1c_ref[...], preferred_element_type=f32))
    z = jnp.dot(_gelu(t), wm2_ref[...], preferred_element_type=f32) + bm2_ref[...]
    dw = jnp.mean(jax.nn.sigmoid(qd_ref[...] * wd_ref[...] + bd_ref[...]),
                  axis=-1, keepdims=True)
    m_out[...] = z * (1.0 + dw)


def _k5(ad, bs, gate_embed, qdist, p):
    spec_p = pl.BlockSpec((BE, 2 * H), lambda i: (i, 0))
    spec_e = pl.BlockSpec((BE, H), lambda i: (i, 0))
    spec_t = pl.BlockSpec((BE, 1), lambda i: (i, 0))
    spec_w1 = pl.BlockSpec((H, 2 * H), lambda i: (0, 0))
    spec_w2 = pl.BlockSpec((2 * H, H), lambda i: (0, 0))
    spec_bh = pl.BlockSpec((1, H), lambda i: (0, 0))
    spec_b4 = pl.BlockSpec((1, HEADS), lambda i: (0, 0))
    return pl.pallas_call(
        _k5_body,
        grid=(GE,),
        in_specs=[spec_p, spec_p, spec_e, spec_t, spec_w1, spec_w2, spec_bh,
                  spec_b4, spec_b4],
        out_specs=spec_e,
        out_shape=jax.ShapeDtypeStruct((E, H), jnp.float32),
    )(ad, bs, gate_embed, qdist.reshape(E, 1), p['Wm1'][2 * H:], p['Wm2'],
      p['bm2'].reshape(1, H), p['Wd'], p['bd'].reshape(1, HEADS))


def _k6_body(h_ref, a2p_ref, wg1_ref, wg2_ref, bg_ref, wu1_ref, wu2_ref,
             bu_ref, lcg_ref, lcb_ref, n1g_ref, n1b_ref, wf1_ref, bf1_ref,
             wf2_ref, bf2_ref, wga1_ref, wga2_ref, bga_ref, n2g_ref,
             n2b_ref, out_ref):
    f32 = jnp.float32
    h = h_ref[...]
    agg2 = a2p_ref[0] + a2p_ref[1]
    gate = jax.nn.sigmoid(
        jnp.dot(h, wg1_ref[...], preferred_element_type=f32)
        + jnp.dot(agg2, wg2_ref[...], preferred_element_type=f32) + bg_ref[...])
    upd = _gelu(
        jnp.dot(h, wu1_ref[...], preferred_element_type=f32)
        + jnp.dot(agg2, wu2_ref[...], preferred_element_type=f32) + bu_ref[...])
    hc = _ln(h * (1.0 - gate) + upd * gate, lcg_ref[...], lcb_ref[...])
    hn = _ln(hc, n1g_ref[...], n1b_ref[...])
    hff = jnp.dot(_gelu(jnp.dot(hn, wf1_ref[...], preferred_element_type=f32)
                        + bf1_ref[...]), wf2_ref[...],
                  preferred_element_type=f32) + bf2_ref[...]
    g2 = jax.nn.sigmoid(
        jnp.dot(hc, wga1_ref[...], preferred_element_type=f32)
        + jnp.dot(hff, wga2_ref[...], preferred_element_type=f32) + bga_ref[...])
    out_ref[...] = _ln(hc + g2 * hff, n2g_ref[...], n2b_ref[...])


def _k6(h, agg2_part, p):
    spec_n = pl.BlockSpec((BN, H), lambda i: (i, 0))
    spec_p = pl.BlockSpec((2, BN, H), lambda i: (0, i, 0))
    spec_w = pl.BlockSpec((H, H), lambda i: (0, 0))
    spec_wf1 = pl.BlockSpec((H, FF), lambda i: (0, 0))
    spec_wf2 = pl.BlockSpec((FF, H), lambda i: (0, 0))
    spec_b = pl.BlockSpec((1, H), lambda i: (0, 0))
    spec_bf = pl.BlockSpec((1, FF), lambda i: (0, 0))
    return pl.pallas_call(
        _k6_body,
        grid=(GN,),
        in_specs=[spec_n, spec_p, spec_w, spec_w, spec_b, spec_w, spec_w,
                  spec_b, spec_b, spec_b, spec_b, spec_b, spec_wf1, spec_bf,
                  spec_wf2, spec_b, spec_w, spec_w, spec_b, spec_b, spec_b],
        out_specs=spec_n,
        out_shape=jax.ShapeDtypeStruct((N, H), jnp.float32),
    )(h, agg2_part,
      p['Wg'][:H], p['Wg'][H:], p['bg'].reshape(1, H),
      p['Wu'][:H], p['Wu'][H:], p['bu'].reshape(1, H),
      p['ln_c_g'].reshape(1, H), p['ln_c_b'].reshape(1, H),
      p['n1_g'].reshape(1, H), p['n1_b'].reshape(1, H),
      p['Wf1'], p['bf1'].reshape(1, FF), p['Wf2'], p['bf2'].reshape(1, H),
      p['Wgate'][:H], p['Wgate'][H:], p['bgate'].reshape(1, H),
      p['n2_g'].reshape(1, H), p['n2_b'].reshape(1, H))


def kernel(x, edge_index, gate_embed, temporal_pos, qubit_distance, params):
    p = params
    src = edge_index[0]
    dst = edge_index[1]
    zmsg = jnp.zeros((N, H), jnp.float32)

    xq, xk, xv = _k1(x, p)
    qd, ks, vs = _g1(xq, xk, xv, dst, src)
    msg, exw = _k2(qd, ks, vs, gate_embed, temporal_pos, p)
    agg_part = _sseg(msg, dst, zmsg, H)
    den_part = _sseg(exw, dst, zmsg, H)
    h, a, b = _k4(agg_part, den_part, x, p)
    ad, bs = _g2(a, b, dst, src)
    msg2 = _k5(ad, bs, gate_embed, qubit_distance, p)
    agg2_part = _sseg(msg2, dst, zmsg, H)
    return _k6(h, agg2_part, p)
